# baseline (speedup 1.0000x reference)
import numpy as np
import jax
import jax.numpy as jnp
from jax.experimental import pallas as pl
from jax.experimental.pallas import tpu as pltpu

_D_MODEL = 2048
_BS = 256


def _rotation_tables(seq_len):
    D = _D_MODEL
    pairfreq = jnp.exp(
        jnp.arange(0, D, 2, dtype=jnp.float32) * (-np.log(10000.0) / D)
    )
    g = jnp.repeat(pairfreq, 2)[None, :]
    r = jnp.arange(_BS, dtype=jnp.float32)[:, None]
    t_sin, t_cos = jnp.sin(r * g), jnp.cos(r * g)
    nblk = seq_len // _BS
    base = (jnp.arange(nblk, dtype=jnp.float32) * _BS)[:, None]
    sb, cb = jnp.sin(base * g), jnp.cos(base * g)
    even = (jnp.arange(D) % 2 == 0)[None, :]
    p = jnp.where(even, sb, cb).reshape(nblk, 1, D)
    q = jnp.where(even, cb, -sb).reshape(nblk, 1, D)
    return t_sin, t_cos, p, q


def _body(mw_ref, x_ref, learn_ref, tsin_ref, tcos_ref, p_ref, q_ref, o_ref):
    w = jax.nn.sigmoid(mw_ref[0, 0])
    wp, wq, wm = w * p_ref[0], w * q_ref[0], 1.0 - w
    for r in range(0, _BS, 8):
        sl = pl.ds(r, 8)
        comb = wp * tcos_ref[sl, :] + wq * tsin_ref[sl, :] \
            + wm * learn_ref[sl, :]
        for b in range(x_ref.shape[0]):
            o_ref[b, sl, :] = x_ref[b, sl, :] + comb


def kernel(x, pe_learn, mix_weight):
    B, S, D = x.shape
    mw = jnp.asarray(mix_weight, jnp.float32).reshape(1, 1)
    t_sin, t_cos, p, q = _rotation_tables(S)
    return pl.pallas_call(
        _body,
        grid=(S // _BS,),
        in_specs=[
            pl.BlockSpec(memory_space=pltpu.SMEM),
            pl.BlockSpec((B, _BS, D), lambda i: (0, i, 0)),
            pl.BlockSpec((_BS, D), lambda i: (i, 0)),
            pl.BlockSpec((_BS, D), lambda i: (0, 0)),
            pl.BlockSpec((_BS, D), lambda i: (0, 0)),
            pl.BlockSpec((1, 1, D), lambda i: (i, 0, 0)),
            pl.BlockSpec((1, 1, D), lambda i: (i, 0, 0)),
        ],
        out_specs=pl.BlockSpec((B, _BS, D), lambda i: (0, i, 0)),
        out_shape=jax.ShapeDtypeStruct((B, S, D), x.dtype),
        compiler_params=pltpu.CompilerParams(
            dimension_semantics=("parallel",),
        ),
    )(mw, x, pe_learn, t_sin, t_cos, p, q)
